# baseline (speedup 1.0000x reference)
import functools

import jax
import jax.numpy as jnp
from jax import lax
from jax.experimental import pallas as pl
from jax.experimental.pallas import tpu as pltpu
from jax.experimental.pallas import tpu_sc as plsc

_B, _Q, _D = 8, 1024, 64
_K = 8192
_BQ = _B * _Q

_ROWS = 1024
_KC = 2048


_LG = _KC // 128


def _argmin_body(x_ref, cb_ref, iota_ref, idx_ref, d2_ref):
    x = x_ref[...]
    x2 = jnp.sum(x * x, axis=1, keepdims=True)
    s = x2 + jnp.float32(_D)
    xd = x + x

    acc = None
    for j in range(_K // _KC):
        c = cb_ref[pl.ds(j * _KC, _KC), :]
        xc2 = lax.dot_general(xd, c, (((1,), (1,)), ((), ())),
                              preferred_element_type=jnp.float32)
        d2 = s - xc2
        d2_ref[:, pl.ds(j * _KC, _KC)] = d2
        for g in range(_LG):
            blk = d2[:, g * 128:(g + 1) * 128]
            acc = blk if acc is None else jnp.minimum(acc, blk)
    m2 = jnp.min(acc, axis=1, keepdims=True)

    m2c = jnp.maximum(m2, 0.0)
    dmin = jnp.sqrt(m2c)
    t0b = lax.bitcast_convert_type(dmin * dmin, jnp.int32)
    thr = m2c
    for jj in range(3):
        t = lax.bitcast_convert_type(t0b + jj, jnp.float32)
        thr = jnp.where(jnp.sqrt(t) == dmin, jnp.maximum(thr, t), thr)

    iacc = None
    for j in range(_K // _KC):
        d2 = d2_ref[:, pl.ds(j * _KC, _KC)]
        ki = iota_ref[:, pl.ds(j * _KC, _KC)]
        cand = jnp.where(d2 <= thr, ki, jnp.float32(_K))
        for g in range(_LG):
            blk = cand[:, g * 128:(g + 1) * 128]
            iacc = blk if iacc is None else jnp.minimum(iacc, blk)
    idx_ref[...] = jnp.min(iacc, axis=1, keepdims=True).astype(jnp.int32)


_tc_argmin = pl.pallas_call(
    _argmin_body,
    grid=(_BQ // _ROWS,),
    in_specs=[
        pl.BlockSpec((_ROWS, _D), lambda i: (i, 0)),
        pl.BlockSpec((_K, _D), lambda i: (0, 0)),
        pl.BlockSpec((1, _K), lambda i: (0, 0)),
    ],
    out_specs=pl.BlockSpec((_ROWS, 1), lambda i: (i, 0)),
    out_shape=jax.ShapeDtypeStruct((_BQ, 1), jnp.int32),
    scratch_shapes=[pltpu.VMEM((_ROWS, _K), jnp.float32)],
)


_ICHUNK = 128
_DPAD = 128


@functools.lru_cache(maxsize=None)
def _make_sc_gather():
    info = plsc.get_sparse_core_info()
    nc, ns = info.num_cores, info.num_subcores
    nw = nc * ns
    bpw = _BQ // nw
    ni = bpw // _ICHUNK
    mesh = plsc.VectorSubcoreMesh(core_axis_name="c", subcore_axis_name="s")

    @functools.partial(
        pl.kernel,
        mesh=mesh,
        out_type=jax.ShapeDtypeStruct((_BQ, _DPAD), jnp.float32),
        scratch_types=[
            pltpu.VMEM((ni, _ICHUNK), jnp.int32),
            pltpu.VMEM((bpw, _DPAD), jnp.float32),
            pltpu.SemaphoreType.DMA,
        ],
    )
    def _sc_gather(table_hbm, idx_hbm, out_hbm, idx_v, rows_v, sem):
        wid = lax.axis_index("s") * nc + lax.axis_index("c")
        base = wid * bpw
        pltpu.sync_copy(idx_hbm.at[pl.ds(wid * ni, ni)], idx_v)
        copies = [
            pltpu.async_copy(table_hbm.at[idx_v.at[j]],
                             rows_v.at[pl.ds(j * _ICHUNK, _ICHUNK)], sem)
            for j in range(ni)
        ]
        for cp in copies:
            cp.wait()
        pltpu.sync_copy(rows_v, out_hbm.at[pl.ds(base, bpw)])

    return _sc_gather


def kernel(signal, codebook):
    x = signal.reshape(_BQ, _D)
    kiota = jnp.arange(_K, dtype=jnp.float32).reshape(1, _K)
    idx = _tc_argmin(x, codebook, kiota).reshape(_BQ // _ICHUNK, _ICHUNK)
    cb_pad = jnp.pad(codebook, ((0, 0), (0, _DPAD - _D)))
    rows = _make_sc_gather()(cb_pad, idx)
    return rows[:, :_D].reshape(_B, _Q, _D)


# speedup vs baseline: 1.0210x; 1.0210x over previous
import functools

import jax
import jax.numpy as jnp
from jax import lax
from jax.experimental import pallas as pl
from jax.experimental.pallas import tpu as pltpu
from jax.experimental.pallas import tpu_sc as plsc

_B, _Q, _D = 8, 1024, 64
_K = 8192
_BQ = _B * _Q

_ROWS = 1024
_KC = 2048
_LG = _KC // 128

_ICHUNK = 128
_DPAD = 128


def _argmin_body(x_ref, cb_ref, iota_ref, idx_ref, cbp_ref, d2_ref):
    @pl.when(pl.program_id(0) == 0)
    def _pad_codebook():
        cbp_ref[:, 0:_D] = cb_ref[...]
        cbp_ref[:, _D:_DPAD] = jnp.zeros((_K, _DPAD - _D), jnp.float32)

    x = x_ref[...]
    x2 = jnp.sum(x * x, axis=1, keepdims=True)
    s = x2 + jnp.float32(_D)
    xd = x + x

    acc = None
    for j in range(_K // _KC):
        c = cb_ref[pl.ds(j * _KC, _KC), :]
        xc2 = lax.dot_general(xd, c, (((1,), (1,)), ((), ())),
                              preferred_element_type=jnp.float32)
        d2 = s - xc2
        d2_ref[:, pl.ds(j * _KC, _KC)] = d2
        for g in range(_LG):
            blk = d2[:, g * 128:(g + 1) * 128]
            acc = blk if acc is None else jnp.minimum(acc, blk)
    m2 = jnp.min(acc, axis=1, keepdims=True)

    m2c = jnp.maximum(m2, 0.0)
    dmin = jnp.sqrt(m2c)
    t0b = lax.bitcast_convert_type(dmin * dmin, jnp.int32)
    thr = m2c
    for jj in range(3):
        t = lax.bitcast_convert_type(t0b + jj, jnp.float32)
        thr = jnp.where(jnp.sqrt(t) == dmin, jnp.maximum(thr, t), thr)

    iacc = None
    for j in range(_K // _KC):
        d2 = d2_ref[:, pl.ds(j * _KC, _KC)]
        ki = iota_ref[:, pl.ds(j * _KC, _KC)]
        cand = jnp.where(d2 <= thr, ki, jnp.float32(_K))
        for g in range(_LG):
            blk = cand[:, g * 128:(g + 1) * 128]
            iacc = blk if iacc is None else jnp.minimum(iacc, blk)
    idx_ref[...] = jnp.min(iacc, axis=1, keepdims=True).astype(jnp.int32)


_tc_argmin = pl.pallas_call(
    _argmin_body,
    grid=(_BQ // _ROWS,),
    in_specs=[
        pl.BlockSpec((_ROWS, _D), lambda i: (i, 0)),
        pl.BlockSpec((_K, _D), lambda i: (0, 0)),
        pl.BlockSpec((1, _K), lambda i: (0, 0)),
    ],
    out_specs=[pl.BlockSpec((_ROWS, 1), lambda i: (i, 0)),
               pl.BlockSpec((_K, _DPAD), lambda i: (0, 0))],
    out_shape=[jax.ShapeDtypeStruct((_BQ, 1), jnp.int32),
               jax.ShapeDtypeStruct((_K, _DPAD), jnp.float32)],
    scratch_shapes=[pltpu.VMEM((_ROWS, _K), jnp.float32)],
)


@functools.lru_cache(maxsize=None)
def _make_sc_gather():
    info = plsc.get_sparse_core_info()
    nc, ns = info.num_cores, info.num_subcores
    nw = nc * ns
    bpw = _BQ // nw
    ni = bpw // _ICHUNK
    mesh = plsc.VectorSubcoreMesh(core_axis_name="c", subcore_axis_name="s")

    @functools.partial(
        pl.kernel,
        mesh=mesh,
        out_type=jax.ShapeDtypeStruct((_BQ, _DPAD), jnp.float32),
        scratch_types=[
            pltpu.VMEM((ni, _ICHUNK), jnp.int32),
            pltpu.VMEM((bpw, _DPAD), jnp.float32),
            pltpu.SemaphoreType.DMA,
        ],
    )
    def _sc_gather(table_hbm, idx_hbm, out_hbm, idx_v, rows_v, sem):
        wid = lax.axis_index("s") * nc + lax.axis_index("c")
        base = wid * bpw
        pltpu.sync_copy(idx_hbm.at[pl.ds(wid * ni, ni)], idx_v)
        copies = [
            pltpu.async_copy(table_hbm.at[idx_v.at[j]],
                             rows_v.at[pl.ds(j * _ICHUNK, _ICHUNK)], sem)
            for j in range(ni)
        ]
        for cp in copies:
            cp.wait()
        pltpu.sync_copy(rows_v, out_hbm.at[pl.ds(base, bpw)])

    return _sc_gather


def kernel(signal, codebook):
    x = signal.reshape(_BQ, _D)
    kiota = jnp.arange(_K, dtype=jnp.float32).reshape(1, _K)
    idx, cb_pad = _tc_argmin(x, codebook, kiota)
    rows = _make_sc_gather()(cb_pad, idx.reshape(_BQ // _ICHUNK, _ICHUNK))
    return rows[:, :_D].reshape(_B, _Q, _D)
